# baseline (speedup 1.0000x reference)
import functools

import jax
import jax.numpy as jnp
from jax import lax
from jax.experimental import pallas as pl
from jax.experimental.pallas import tpu as pltpu
from jax.experimental.pallas import tpu_sc as plsc

B, S, D_MODEL, D_FF, E, TOP_K = 2, 2048, 768, 3072, 8, 2
T = B * S
NSLOT = T * TOP_K
BLK = 256
NB = NSLOT // BLK + E
NROWS = NB * BLK

NC, NS = 2, 16
NW = NC * NS
TPW = T // NW
SPW = NSLOT // NW
SUB = 64


def _routing_body(x_ref, gw_ref, gb_ref, dest_ref, w_ref, be_ref):
    f32 = jnp.float32
    logits = lax.dot_general(gw_ref[...], x_ref[...],
                             (((1,), (1,)), ((), ())),
                             preferred_element_type=f32)
    logits = logits + gb_ref[...]
    m = jnp.max(logits, axis=0, keepdims=True)
    p = jnp.exp(logits - m)
    p = p / jnp.sum(p, axis=0, keepdims=True)

    e_iota = lax.broadcasted_iota(jnp.int32, (E, T), 0)
    m0 = jnp.max(p, axis=0, keepdims=True)
    i0 = jnp.min(jnp.where(p == m0, e_iota, E), axis=0, keepdims=True)
    oh0 = e_iota == i0
    p1 = jnp.where(oh0, -1.0, p)
    m1 = jnp.max(p1, axis=0, keepdims=True)
    i1 = jnp.min(jnp.where(p1 == m1, e_iota, E), axis=0, keepdims=True)
    oh1 = e_iota == i1
    zsum = m0 + m1

    ohs = jnp.concatenate([oh0.astype(f32), oh1.astype(f32)], axis=1)
    c = ohs
    d = 1
    while d < NSLOT:
        c = c + jnp.concatenate(
            [jnp.zeros((E, d), f32), c[:, : NSLOT - d]], axis=1)
        d *= 2
    counts = c[:, NSLOT - 1 : NSLOT]
    nblk = jnp.floor((counts + (BLK - 1)) * (1.0 / BLK))
    inc = nblk
    d = 1
    while d < E:
        inc = inc + jnp.concatenate(
            [jnp.zeros((d, 1), f32), inc[: E - d]], axis=0)
        d *= 2
    excl = inc - nblk
    rowoff = excl * float(BLK)

    rank = c - ohs
    destf = jnp.sum(ohs * (rank + rowoff), axis=0, keepdims=True)
    dest_ref[...] = jnp.concatenate(
        [destf[:, :T], destf[:, T:]], axis=0).astype(jnp.int32)
    w_ref[...] = jnp.concatenate([m0 / zsum, m1 / zsum], axis=0)

    jio = lax.broadcasted_iota(jnp.int32, (1, 128), 1).astype(f32)
    be_ref[...] = (jnp.sum((jio >= excl).astype(jnp.int32),
                           axis=0, keepdims=True) - 1)


def _routing(x2, gate_W, gate_b2):
    return pl.pallas_call(
        _routing_body,
        out_shape=[
            jax.ShapeDtypeStruct((2, T), jnp.int32),
            jax.ShapeDtypeStruct((2, T), jnp.float32),
            jax.ShapeDtypeStruct((1, 128), jnp.int32),
        ],
    )(x2, gate_W, gate_b2)


@functools.cache
def _make_dispatch():
    mesh = plsc.VectorSubcoreMesh(core_axis_name="c", subcore_axis_name="s")

    @functools.partial(
        pl.kernel,
        mesh=mesh,
        out_type=jax.ShapeDtypeStruct((NROWS, D_MODEL), jnp.float32),
        scratch_types=[
            pltpu.VMEM((2, TPW), jnp.int32),
            pltpu.VMEM((TPW, D_MODEL), jnp.float32),
            pltpu.SemaphoreType.DMA,
        ],
    )
    def dispatch_k(x_hbm, dest_hbm, xs_hbm, idx_v, xv, sem):
        wid = lax.axis_index("s") * NC + lax.axis_index("c")
        base = wid * TPW
        pltpu.sync_copy(dest_hbm.at[:, pl.ds(base, TPW)], idx_v)
        pltpu.sync_copy(x_hbm.at[pl.ds(base, TPW)], xv)
        pltpu.async_copy(xv, xs_hbm.at[idx_v.at[0]], sem).wait()
        pltpu.async_copy(xv, xs_hbm.at[idx_v.at[1]], sem).wait()

    return dispatch_k


def _ffn_body(be_ref, x_ref, w1_ref, b1_ref, w2_ref, b2_ref, o_ref):
    del be_ref
    f32 = jnp.float32
    h = lax.dot_general(x_ref[...].astype(jnp.bfloat16), w1_ref[0],
                        (((1,), (1,)), ((), ())),
                        preferred_element_type=f32)
    h = h + b1_ref[0]
    h = 0.5 * h * (1.0 + lax.erf(h * 0.7071067811865476))
    o = lax.dot_general(h.astype(jnp.bfloat16), w2_ref[0],
                        (((1,), (1,)), ((), ())),
                        preferred_element_type=f32)
    o_ref[...] = o + b2_ref[0]


def _ffn(xs, W1, b1, W2, b2, blkex):
    grid_spec = pltpu.PrefetchScalarGridSpec(
        num_scalar_prefetch=1,
        grid=(NB,),
        in_specs=[
            pl.BlockSpec((BLK, D_MODEL), lambda i, be: (i, 0)),
            pl.BlockSpec((1, D_FF, D_MODEL), lambda i, be: (be[i], 0, 0)),
            pl.BlockSpec((1, 1, D_FF), lambda i, be: (be[i], 0, 0)),
            pl.BlockSpec((1, D_MODEL, D_FF), lambda i, be: (be[i], 0, 0)),
            pl.BlockSpec((1, 1, D_MODEL), lambda i, be: (be[i], 0, 0)),
        ],
        out_specs=pl.BlockSpec((BLK, D_MODEL), lambda i, be: (i, 0)),
    )
    return pl.pallas_call(
        _ffn_body,
        grid_spec=grid_spec,
        out_shape=jax.ShapeDtypeStruct((NROWS, D_MODEL), jnp.float32),
    )(blkex, xs, W1.astype(jnp.bfloat16), b1.reshape(E, 1, D_FF),
      W2.astype(jnp.bfloat16), b2.reshape(E, 1, D_MODEL))


@functools.cache
def _make_combine_gather():
    mesh = plsc.VectorSubcoreMesh(core_axis_name="c", subcore_axis_name="s")

    @functools.partial(
        pl.kernel,
        mesh=mesh,
        out_type=jax.ShapeDtypeStruct((NSLOT, D_MODEL), jnp.float32),
        scratch_types=[
            pltpu.VMEM((SPW // SUB, SUB), jnp.int32),
            pltpu.VMEM((SUB, D_MODEL), jnp.float32),
            pltpu.SemaphoreType.DMA,
        ],
    )
    def combine_k(ys_hbm, dest3_hbm, g_hbm, idx_v, buf_v, sem):
        wid = lax.axis_index("s") * NC + lax.axis_index("c")
        nsub = SPW // SUB
        pltpu.sync_copy(dest3_hbm.at[pl.ds(wid * nsub, nsub)], idx_v)
        for i in range(nsub):
            pltpu.async_copy(ys_hbm.at[idx_v.at[i]], buf_v, sem).wait()
            pltpu.sync_copy(buf_v, g_hbm.at[pl.ds(wid * SPW + i * SUB, SUB)])

    return combine_k


BE_ROWS = 1024


def _wsum_body(g0_ref, g1_ref, w_ref, o_ref):
    f32 = jnp.float32
    idr = lax.broadcasted_iota(jnp.int32, (BE_ROWS, BE_ROWS), 0)
    idc = lax.broadcasted_iota(jnp.int32, (BE_ROWS, BE_ROWS), 1)
    eyem = (idr == idc).astype(f32)
    wcols = lax.dot_general(eyem, w_ref[...], (((1,), (1,)), ((), ())),
                            preferred_element_type=f32)
    o_ref[...] = (wcols[:, 0:1] * g0_ref[...] +
                  wcols[:, 1:2] * g1_ref[...])


def _wsum(g, w01):
    nblk = T // BE_ROWS
    return pl.pallas_call(
        _wsum_body,
        grid=(nblk,),
        in_specs=[
            pl.BlockSpec((BE_ROWS, D_MODEL), lambda i: (i, 0)),
            pl.BlockSpec((BE_ROWS, D_MODEL), lambda i: (i + nblk, 0)),
            pl.BlockSpec((2, BE_ROWS), lambda i: (0, i)),
        ],
        out_specs=pl.BlockSpec((BE_ROWS, D_MODEL), lambda i: (i, 0)),
        out_shape=jax.ShapeDtypeStruct((T, D_MODEL), jnp.float32),
    )(g, g, w01)


def kernel(x, gate_W, gate_b, W1, b1, W2, b2):
    x2 = x.reshape(T, D_MODEL)
    dest01, w01, blkex = _routing(x2, gate_W, gate_b.reshape(E, 1))
    xs = _make_dispatch()(x2, dest01)
    ys = _ffn(xs, W1, b1, W2, b2, blkex[0])
    g = _make_combine_gather()(ys, dest01.reshape(NSLOT // SUB, SUB))
    out = _wsum(g, w01)
    return out.reshape(B, S, D_MODEL)


# speedup vs baseline: 1.2597x; 1.2597x over previous
import functools

import jax
import jax.numpy as jnp
from jax import lax
from jax.experimental import pallas as pl
from jax.experimental.pallas import tpu as pltpu
from jax.experimental.pallas import tpu_sc as plsc

B, S, D_MODEL, D_FF, E, TOP_K = 2, 2048, 768, 3072, 8, 2
T = B * S
NSLOT = T * TOP_K
BLK = 256
NB = NSLOT // BLK + E
NROWS = NB * BLK

NC, NS = 2, 16
NW = NC * NS
TPW = T // NW
SPW = NSLOT // NW
SUB = 64


def _routing_body(x_ref, gw_ref, gb_ref, dest_ref, w_ref, be_ref):
    f32 = jnp.float32
    logits = lax.dot_general(gw_ref[...], x_ref[...],
                             (((1,), (1,)), ((), ())),
                             preferred_element_type=f32)
    logits = logits + gb_ref[...]
    m = jnp.max(logits, axis=0, keepdims=True)
    p = jnp.exp(logits - m)
    p = p / jnp.sum(p, axis=0, keepdims=True)

    e_iota = lax.broadcasted_iota(jnp.int32, (E, T), 0)
    m0 = jnp.max(p, axis=0, keepdims=True)
    i0 = jnp.min(jnp.where(p == m0, e_iota, E), axis=0, keepdims=True)
    oh0 = e_iota == i0
    p1 = jnp.where(oh0, -1.0, p)
    m1 = jnp.max(p1, axis=0, keepdims=True)
    i1 = jnp.min(jnp.where(p1 == m1, e_iota, E), axis=0, keepdims=True)
    oh1 = e_iota == i1
    zsum = m0 + m1

    ohs = jnp.concatenate([oh0.astype(f32), oh1.astype(f32)], axis=1)
    c = ohs
    d = 1
    while d < NSLOT:
        c = c + jnp.concatenate(
            [jnp.zeros((E, d), f32), c[:, : NSLOT - d]], axis=1)
        d *= 2
    counts = c[:, NSLOT - 1 : NSLOT]
    nblk = jnp.floor((counts + (BLK - 1)) * (1.0 / BLK))
    inc = nblk
    d = 1
    while d < E:
        inc = inc + jnp.concatenate(
            [jnp.zeros((d, 1), f32), inc[: E - d]], axis=0)
        d *= 2
    excl = inc - nblk
    rowoff = excl * float(BLK)

    rank = c - ohs
    destf = jnp.sum(ohs * (rank + rowoff), axis=0, keepdims=True)
    dest_ref[...] = jnp.concatenate(
        [destf[:, :T], destf[:, T:]], axis=0).astype(jnp.int32)
    w_ref[...] = jnp.concatenate([m0 / zsum, m1 / zsum], axis=0)

    jio = lax.broadcasted_iota(jnp.int32, (1, 128), 1).astype(f32)
    be_ref[...] = (jnp.sum((jio >= excl).astype(jnp.int32),
                           axis=0, keepdims=True) - 1)


def _routing(x2, gate_W, gate_b2):
    return pl.pallas_call(
        _routing_body,
        out_shape=[
            jax.ShapeDtypeStruct((2, T), jnp.int32),
            jax.ShapeDtypeStruct((2, T), jnp.float32),
            jax.ShapeDtypeStruct((1, 128), jnp.int32),
        ],
    )(x2, gate_W, gate_b2)


@functools.cache
def _make_dispatch():
    mesh = plsc.VectorSubcoreMesh(core_axis_name="c", subcore_axis_name="s")

    @functools.partial(
        pl.kernel,
        mesh=mesh,
        out_type=jax.ShapeDtypeStruct((NROWS, D_MODEL), jnp.float32),
        scratch_types=[
            pltpu.VMEM((2, TPW), jnp.int32),
            pltpu.VMEM((TPW, D_MODEL), jnp.float32),
            pltpu.SemaphoreType.DMA,
        ],
    )
    def dispatch_k(x_hbm, dest_hbm, xs_hbm, idx_v, xv, sem):
        wid = lax.axis_index("s") * NC + lax.axis_index("c")
        base = wid * TPW
        pltpu.sync_copy(dest_hbm.at[:, pl.ds(base, TPW)], idx_v)
        pltpu.sync_copy(x_hbm.at[pl.ds(base, TPW)], xv)
        pltpu.async_copy(xv, xs_hbm.at[idx_v.at[0]], sem).wait()
        pltpu.async_copy(xv, xs_hbm.at[idx_v.at[1]], sem).wait()

    return dispatch_k


def _ffn_body(be_ref, x_ref, w1_ref, b1_ref, w2_ref, b2_ref, o_ref):
    del be_ref
    f32 = jnp.float32
    h = lax.dot_general(x_ref[...], w1_ref[0], (((1,), (1,)), ((), ())),
                        preferred_element_type=f32)
    h = h + b1_ref[0]
    h = 0.5 * h * (1.0 + lax.erf(h * 0.7071067811865476))
    o = lax.dot_general(h, w2_ref[0], (((1,), (1,)), ((), ())),
                        preferred_element_type=f32)
    o_ref[...] = o + b2_ref[0]


def _ffn(xs, W1, b1, W2, b2, blkex):
    grid_spec = pltpu.PrefetchScalarGridSpec(
        num_scalar_prefetch=1,
        grid=(NB,),
        in_specs=[
            pl.BlockSpec((BLK, D_MODEL), lambda i, be: (i, 0)),
            pl.BlockSpec((1, D_FF, D_MODEL), lambda i, be: (be[i], 0, 0)),
            pl.BlockSpec((1, 1, D_FF), lambda i, be: (be[i], 0, 0)),
            pl.BlockSpec((1, D_MODEL, D_FF), lambda i, be: (be[i], 0, 0)),
            pl.BlockSpec((1, 1, D_MODEL), lambda i, be: (be[i], 0, 0)),
        ],
        out_specs=pl.BlockSpec((BLK, D_MODEL), lambda i, be: (i, 0)),
    )
    return pl.pallas_call(
        _ffn_body,
        grid_spec=grid_spec,
        out_shape=jax.ShapeDtypeStruct((NROWS, D_MODEL), jnp.float32),
    )(blkex, xs, W1, b1.reshape(E, 1, D_FF), W2, b2.reshape(E, 1, D_MODEL))


NHALF = TPW // SUB


@functools.cache
def _make_combine_wsum():
    mesh = plsc.VectorSubcoreMesh(core_axis_name="c", subcore_axis_name="s")

    @functools.partial(
        pl.kernel,
        mesh=mesh,
        out_type=jax.ShapeDtypeStruct((T, D_MODEL), jnp.float32),
        scratch_types=[
            pltpu.VMEM((2, NHALF, SUB), jnp.int32),
            pltpu.VMEM((TPW,), jnp.float32),
            pltpu.VMEM((TPW,), jnp.float32),
            pltpu.VMEM((SUB, D_MODEL), jnp.float32),
            pltpu.VMEM((SUB, D_MODEL), jnp.float32),
            pltpu.SemaphoreType.DMA,
        ],
    )
    def combine_k(ys_hbm, dest4_hbm, w_hbm, o_hbm, idx_v, w0_v, w1_v, b0, b1,
                  sem):
        wid = lax.axis_index("s") * NC + lax.axis_index("c")
        base = wid * TPW
        pltpu.sync_copy(dest4_hbm.at[:, pl.ds(wid * NHALF, NHALF)], idx_v)
        pltpu.sync_copy(w_hbm.at[0, pl.ds(base, TPW)], w0_v)
        pltpu.sync_copy(w_hbm.at[1, pl.ds(base, TPW)], w1_v)
        for half in range(NHALF):
            pltpu.async_copy(ys_hbm.at[idx_v.at[0, half]], b0, sem).wait()
            pltpu.async_copy(ys_hbm.at[idx_v.at[1, half]], b1, sem).wait()

            zero16 = jnp.zeros((16,), jnp.int32)

            def grp_body(g, _, half=half):
                off = pl.multiple_of(half * SUB + g * 16, 16)
                sv0 = w0_v[pl.ds(off, 16)]
                sv1 = w1_v[pl.ds(off, 16)]

                def row_body(r16, lane):
                    gdn = lax.GatherDimensionNumbers(
                        offset_dims=(), collapsed_slice_dims=(0,),
                        start_index_map=(0,))
                    s0 = lax.gather(
                        sv0, lane[:, None], gdn, (1,),
                        mode=lax.GatherScatterMode.PROMISE_IN_BOUNDS)
                    s1 = lax.gather(
                        sv1, lane[:, None], gdn, (1,),
                        mode=lax.GatherScatterMode.PROMISE_IN_BOUNDS)
                    row = g * 16 + r16
                    for c in range(D_MODEL // 16):
                        v0 = b0[row, pl.ds(c * 16, 16)]
                        v1 = b1[row, pl.ds(c * 16, 16)]
                        b0[row, pl.ds(c * 16, 16)] = s0 * v0 + s1 * v1
                    return lane + 1

                lax.fori_loop(0, 16, row_body, zero16)
                return 0

            lax.fori_loop(0, SUB // 16, grp_body, 0)
            pltpu.sync_copy(b0, o_hbm.at[pl.ds(base + half * SUB, SUB)])

    return combine_k


def kernel(x, gate_W, gate_b, W1, b1, W2, b2):
    x2 = x.reshape(T, D_MODEL)
    dest01, w01, blkex = _routing(x2, gate_W, gate_b.reshape(E, 1))
    xs = _make_dispatch()(x2, dest01)
    ys = _ffn(xs, W1, b1, W2, b2, blkex[0])
    out = _make_combine_wsum()(ys, dest01.reshape(2, T // SUB, SUB), w01)
    return out.reshape(B, S, D_MODEL)


# speedup vs baseline: 1.3218x; 1.0493x over previous
import functools

import jax
import jax.numpy as jnp
from jax import lax
from jax.experimental import pallas as pl
from jax.experimental.pallas import tpu as pltpu
from jax.experimental.pallas import tpu_sc as plsc

B, S, D_MODEL, D_FF, E, TOP_K = 2, 2048, 768, 3072, 8, 2
T = B * S
NSLOT = T * TOP_K
BLK = 512
NB = NSLOT // BLK + E
NROWS = NB * BLK

NC, NS = 2, 16
NW = NC * NS
TPW = T // NW
SPW = NSLOT // NW
SUB = 64


def _routing_body(x_ref, gw_ref, gb_ref, dest_ref, w_ref, be_ref):
    f32 = jnp.float32
    logits = lax.dot_general(gw_ref[...], x_ref[...],
                             (((1,), (1,)), ((), ())),
                             preferred_element_type=f32)
    logits = logits + gb_ref[...]
    m = jnp.max(logits, axis=0, keepdims=True)
    p = jnp.exp(logits - m)
    p = p / jnp.sum(p, axis=0, keepdims=True)

    e_iota = lax.broadcasted_iota(jnp.int32, (E, T), 0)
    m0 = jnp.max(p, axis=0, keepdims=True)
    i0 = jnp.min(jnp.where(p == m0, e_iota, E), axis=0, keepdims=True)
    oh0 = e_iota == i0
    p1 = jnp.where(oh0, -1.0, p)
    m1 = jnp.max(p1, axis=0, keepdims=True)
    i1 = jnp.min(jnp.where(p1 == m1, e_iota, E), axis=0, keepdims=True)
    oh1 = e_iota == i1
    zsum = m0 + m1

    ohs = jnp.concatenate([oh0.astype(f32), oh1.astype(f32)], axis=1)
    c = ohs
    d = 1
    while d < NSLOT:
        c = c + jnp.concatenate(
            [jnp.zeros((E, d), f32), c[:, : NSLOT - d]], axis=1)
        d *= 2
    counts = c[:, NSLOT - 1 : NSLOT]
    nblk = jnp.floor((counts + (BLK - 1)) * (1.0 / BLK))
    inc = nblk
    d = 1
    while d < E:
        inc = inc + jnp.concatenate(
            [jnp.zeros((d, 1), f32), inc[: E - d]], axis=0)
        d *= 2
    excl = inc - nblk
    rowoff = excl * float(BLK)

    rank = c - ohs
    destf = jnp.sum(ohs * (rank + rowoff), axis=0, keepdims=True)
    dest_ref[...] = jnp.concatenate(
        [destf[:, :T], destf[:, T:]], axis=0).astype(jnp.int32)
    w_ref[...] = jnp.concatenate([m0 / zsum, m1 / zsum], axis=0)

    jio = lax.broadcasted_iota(jnp.int32, (1, 128), 1).astype(f32)
    bev = jnp.sum((jio >= excl).astype(jnp.int32), axis=0, keepdims=True) - 1
    tot = jnp.sum(nblk).astype(jnp.int32)
    be_ref[...] = jnp.where(jio == float(NB), tot, bev)


def _routing(x2, gate_W, gate_b2):
    return pl.pallas_call(
        _routing_body,
        out_shape=[
            jax.ShapeDtypeStruct((2, T), jnp.int32),
            jax.ShapeDtypeStruct((2, T), jnp.float32),
            jax.ShapeDtypeStruct((1, 128), jnp.int32),
        ],
    )(x2, gate_W, gate_b2)


@functools.cache
def _make_dispatch():
    mesh = plsc.VectorSubcoreMesh(core_axis_name="c", subcore_axis_name="s")

    @functools.partial(
        pl.kernel,
        mesh=mesh,
        out_type=jax.ShapeDtypeStruct((NROWS, D_MODEL), jnp.float32),
        scratch_types=[
            pltpu.VMEM((2, TPW), jnp.int32),
            pltpu.VMEM((TPW, D_MODEL), jnp.float32),
            pltpu.SemaphoreType.DMA,
        ],
    )
    def dispatch_k(x_hbm, dest_hbm, xs_hbm, idx_v, xv, sem):
        wid = lax.axis_index("s") * NC + lax.axis_index("c")
        base = wid * TPW
        pltpu.sync_copy(dest_hbm.at[:, pl.ds(base, TPW)], idx_v)
        pltpu.sync_copy(x_hbm.at[pl.ds(base, TPW)], xv)
        c0 = pltpu.async_copy(xv, xs_hbm.at[idx_v.at[0]], sem)
        c1 = pltpu.async_copy(xv, xs_hbm.at[idx_v.at[1]], sem)
        c0.wait()
        c1.wait()

    return dispatch_k


def _ffn_body(be_ref, x_ref, w1_ref, b1_ref, w2_ref, b2_ref, o_ref):
    @pl.when(pl.program_id(0) < be_ref[NB])
    def _():
        f32 = jnp.float32
        h = lax.dot_general(x_ref[...], w1_ref[0], (((1,), (1,)), ((), ())),
                            preferred_element_type=f32)
        h = h + b1_ref[0]
        h = 0.5 * h * (1.0 + lax.erf(h * 0.7071067811865476))
        o = lax.dot_general(h, w2_ref[0], (((1,), (1,)), ((), ())),
                            preferred_element_type=f32)
        o_ref[...] = o + b2_ref[0]


def _ffn(xs, W1, b1, W2, b2, blkex):
    grid_spec = pltpu.PrefetchScalarGridSpec(
        num_scalar_prefetch=1,
        grid=(NB,),
        in_specs=[
            pl.BlockSpec((BLK, D_MODEL), lambda i, be: (i, 0)),
            pl.BlockSpec((1, D_FF, D_MODEL), lambda i, be: (be[i], 0, 0)),
            pl.BlockSpec((1, 1, D_FF), lambda i, be: (be[i], 0, 0)),
            pl.BlockSpec((1, D_MODEL, D_FF), lambda i, be: (be[i], 0, 0)),
            pl.BlockSpec((1, 1, D_MODEL), lambda i, be: (be[i], 0, 0)),
        ],
        out_specs=pl.BlockSpec((BLK, D_MODEL), lambda i, be: (i, 0)),
    )
    return pl.pallas_call(
        _ffn_body,
        grid_spec=grid_spec,
        out_shape=jax.ShapeDtypeStruct((NROWS, D_MODEL), jnp.float32),
    )(blkex, xs, W1, b1.reshape(E, 1, D_FF), W2, b2.reshape(E, 1, D_MODEL))


CSUB = 32
NRND = TPW // CSUB


@functools.cache
def _make_combine_wsum():
    mesh = plsc.VectorSubcoreMesh(core_axis_name="c", subcore_axis_name="s")

    @functools.partial(
        pl.kernel,
        mesh=mesh,
        out_type=jax.ShapeDtypeStruct((T, D_MODEL), jnp.float32),
        scratch_types=[
            pltpu.VMEM((2, NRND, CSUB), jnp.int32),
            pltpu.VMEM((TPW,), jnp.float32),
            pltpu.VMEM((TPW,), jnp.float32),
            pltpu.VMEM((2, CSUB, D_MODEL), jnp.float32),
            pltpu.VMEM((2, CSUB, D_MODEL), jnp.float32),
            pltpu.SemaphoreType.DMA,
            pltpu.SemaphoreType.DMA,
        ],
    )
    def combine_k(ys_hbm, dest4_hbm, w_hbm, o_hbm, idx_v, w0_v, w1_v,
                  b0, b1, sem0, sem1):
        wid = lax.axis_index("s") * NC + lax.axis_index("c")
        base = wid * TPW
        pltpu.sync_copy(dest4_hbm.at[:, pl.ds(wid * NRND, NRND)], idx_v)
        pltpu.sync_copy(w_hbm.at[0, pl.ds(base, TPW)], w0_v)
        pltpu.sync_copy(w_hbm.at[1, pl.ds(base, TPW)], w1_v)
        sems = (sem0, sem1)

        def fire(rnd):
            pb = rnd % 2
            return (
                pltpu.async_copy(ys_hbm.at[idx_v.at[0, rnd]], b0.at[pb],
                                 sems[pb]),
                pltpu.async_copy(ys_hbm.at[idx_v.at[1, rnd]], b1.at[pb],
                                 sems[pb]),
            )

        zero16 = jnp.zeros((16,), jnp.int32)
        gdn = lax.GatherDimensionNumbers(
            offset_dims=(), collapsed_slice_dims=(0,), start_index_map=(0,))
        pend = fire(0)
        for rnd in range(NRND):
            pb = rnd % 2
            pend[0].wait()
            pend[1].wait()
            if rnd + 1 < NRND:
                pend = fire(rnd + 1)

            def grp_body(g, _, rnd=rnd, pb=pb):
                off = pl.multiple_of(rnd * CSUB + g * 16, 16)
                sv0 = w0_v[pl.ds(off, 16)]
                sv1 = w1_v[pl.ds(off, 16)]

                def row_body(r16, lane):
                    s0 = lax.gather(
                        sv0, lane[:, None], gdn, (1,),
                        mode=lax.GatherScatterMode.PROMISE_IN_BOUNDS)
                    s1 = lax.gather(
                        sv1, lane[:, None], gdn, (1,),
                        mode=lax.GatherScatterMode.PROMISE_IN_BOUNDS)
                    row = g * 16 + r16
                    for c in range(D_MODEL // 16):
                        v0 = b0[pb, row, pl.ds(c * 16, 16)]
                        v1 = b1[pb, row, pl.ds(c * 16, 16)]
                        b0[pb, row, pl.ds(c * 16, 16)] = s0 * v0 + s1 * v1
                    return lane + 1

                lax.fori_loop(0, 16, row_body, zero16)
                return 0

            lax.fori_loop(0, CSUB // 16, grp_body, 0)
            pltpu.sync_copy(b0.at[pb],
                            o_hbm.at[pl.ds(base + rnd * CSUB, CSUB)])

    return combine_k


def kernel(x, gate_W, gate_b, W1, b1, W2, b2):
    x2 = x.reshape(T, D_MODEL)
    dest01, w01, blkex = _routing(x2, gate_W, gate_b.reshape(E, 1))
    xs = _make_dispatch()(x2, dest01)
    ys = _ffn(xs, W1, b1, W2, b2, blkex[0])
    out = _make_combine_wsum()(ys, dest01.reshape(2, T // CSUB, CSUB), w01)
    return out.reshape(B, S, D_MODEL)
